# Initial kernel scaffold; baseline (speedup 1.0000x reference)
import jax
import jax.numpy as jnp
from jax.experimental import pallas as pl


def kernel(coord, edge_index, y, Wxz_w, Wxz_b, Wm, bm, Wu, bu, Why_w, Why_b):
    raise NotImplementedError("write your pallas kernel here")


# speedup vs baseline: 3.8448x; 3.8448x over previous
import functools

import jax
import jax.numpy as jnp
from jax import lax
from jax.experimental import pallas as pl
from jax.experimental.pallas import tpu as pltpu
from jax.experimental.pallas import tpu_sc as plsc

N = 10000
E = 320000
DIM = 128
NC = 2
NS = 16
SPLIT = 5120
SZ1 = N - SPLIT
AGGR = SPLIT + 8
CHUNK = 80
EPT = E // NS
NCHUNK = EPT // CHUNK
RING = 5


def _clamp_dst(dstc_r, b, half_lo, half_sz):
  for k in range(CHUNK // 16):
    dv = dstc_r[b, pl.ds(k * 16, 16)]
    local = dv - half_lo
    ok = (local >= 0) & (local < half_sz)
    dstc_r[b, pl.ds(k * 16, 16)] = jnp.where(ok, local, SPLIT)


def _spmm_body(t_hbm, src_hbm, dst_hbm, z128, out_hbm,
               src_r, dstc_r, rows_r, agg_sh, *sems):
  gsem = sems[0:RING]
  ssem = sems[RING:2 * RING]

  c = lax.axis_index("c")
  s = lax.axis_index("s")
  ebase = s * EPT
  half_sz = jnp.where(c == 0, SPLIT, SZ1)
  half_lo = c * SPLIT

  @pl.when(s == 0)
  def _zero():
    pltpu.sync_copy(z128, agg_sh)

  plsc.subcore_barrier()

  def _load_and_gather(i, b):
    off = ebase + i * CHUNK
    pltpu.sync_copy(src_hbm.at[pl.ds(off, CHUNK)], src_r.at[b])
    pltpu.sync_copy(dst_hbm.at[pl.ds(off, CHUNK)], dstc_r.at[b])
    _clamp_dst(dstc_r, b, half_lo, half_sz)
    pltpu.async_copy(t_hbm.at[src_r.at[b]], rows_r.at[b], gsem[b])

  for b in range(RING):
    _load_and_gather(b, b)

  def _outer(io, carry):
    for b in range(RING):
      i = io * RING + b
      pltpu.make_async_copy(t_hbm.at[src_r.at[b]], rows_r.at[b],
                            gsem[b]).wait()
      pltpu.async_copy(rows_r.at[b], agg_sh.at[dstc_r.at[b]], ssem[b],
                       add=True)

      j = i + RING
      @pl.when(j < NCHUNK)
      def _prefetch():
        pltpu.make_async_copy(rows_r.at[b], agg_sh.at[dstc_r.at[b]],
                              ssem[b]).wait()
        _load_and_gather(j, b)
    return carry

  lax.fori_loop(0, NCHUNK // RING, _outer, 0)

  for b in range(RING):
    pltpu.make_async_copy(rows_r.at[b], agg_sh.at[dstc_r.at[b]],
                          ssem[b]).wait()

  plsc.subcore_barrier()
  _writeout_halves(agg_sh, out_hbm, c, s)


def _writeout_halves(sh, hbm, c, s):
  @pl.when((s == 0) & (c == 0))
  def _wo0():
    pltpu.sync_copy(sh.at[pl.ds(0, SPLIT)], hbm.at[pl.ds(0, SPLIT)])

  @pl.when((s == 0) & (c == 1))
  def _wo1():
    pltpu.sync_copy(sh.at[pl.ds(0, SZ1)], hbm.at[pl.ds(SPLIT, SZ1)])


def _deg_body(dst_hbm, ones_hbm, z128, deg_hbm, dstc_r, ones_v, deg_sh,
              *ssem):
  c = lax.axis_index("c")
  s = lax.axis_index("s")
  ebase = s * EPT
  half_sz = jnp.where(c == 0, SPLIT, SZ1)
  half_lo = c * SPLIT

  @pl.when(s == 0)
  def _zero():
    pltpu.sync_copy(z128, deg_sh)

  pltpu.sync_copy(ones_hbm, ones_v)
  plsc.subcore_barrier()

  def _one_chunk(i, b):
    off = ebase + i * CHUNK
    pltpu.sync_copy(dst_hbm.at[pl.ds(off, CHUNK)], dstc_r.at[b])
    _clamp_dst(dstc_r, b, half_lo, half_sz)
    pltpu.async_copy(ones_v, deg_sh.at[dstc_r.at[b]], ssem[b], add=True)

  for b in range(RING):
    _one_chunk(b, b)

  def _outer(io, carry):
    for b in range(RING):
      i = io * RING + b
      pltpu.make_async_copy(ones_v, deg_sh.at[dstc_r.at[b]], ssem[b]).wait()
      _one_chunk(i, b)
    return carry

  lax.fori_loop(1, NCHUNK // RING, _outer, 0)

  for b in range(RING):
    pltpu.make_async_copy(ones_v, deg_sh.at[dstc_r.at[b]], ssem[b]).wait()

  plsc.subcore_barrier()
  _writeout_halves(deg_sh, deg_hbm, c, s)


def _sc_mesh():
  return plsc.VectorSubcoreMesh(core_axis_name="c", subcore_axis_name="s")


def _make_spmm():
  full = jax.ShapeDtypeStruct((N, DIM), jnp.float32)
  scratch = [
      pltpu.VMEM((RING, CHUNK), jnp.int32),
      pltpu.VMEM((RING, CHUNK), jnp.int32),
      pltpu.VMEM((RING, CHUNK, DIM), jnp.float32),
      pltpu.VMEM_SHARED((AGGR, DIM), jnp.float32),
  ] + [pltpu.SemaphoreType.DMA] * (2 * RING)
  return pl.kernel(_spmm_body, out_type=full, mesh=_sc_mesh(),
                   scratch_types=scratch)


def _make_deg():
  full = jax.ShapeDtypeStruct((N, DIM), jnp.float32)
  scratch = [
      pltpu.VMEM((RING, CHUNK), jnp.int32),
      pltpu.VMEM((CHUNK, DIM), jnp.float32),
      pltpu.VMEM_SHARED((AGGR, DIM), jnp.float32),
  ] + [pltpu.SemaphoreType.DMA] * RING
  return pl.kernel(_deg_body, out_type=full, mesh=_sc_mesh(),
                   scratch_types=scratch)


def _k0_body(coordp, wxzp, bxz, wm0, bm0, h0_o, t0_o):
  h0 = jnp.dot(coordp[...], wxzp[...],
               preferred_element_type=jnp.float32) + bxz[...]
  h0_o[...] = h0
  t0_o[...] = jnp.dot(h0, wm0[...],
                      preferred_element_type=jnp.float32) + bm0[...]


def _update(h, parts, degp, wut, wub, bub):
  inv = 1.0 / jnp.maximum(degp[:, 0:1], 1.0)
  agg = parts[...] * inv
  u = jnp.dot(h[...], wut[...], preferred_element_type=jnp.float32)
  u += jnp.dot(agg, wub[...], preferred_element_type=jnp.float32)
  u = jnp.maximum(u + bub[...], 0.0)
  return h[...] + u


def _upd_body(h, parts, degp, wut, wub, bub, wmn, bmn, hn_o, tn_o):
  hn = _update(h, parts, degp, wut, wub, bub)
  hn_o[...] = hn
  tn_o[...] = jnp.dot(hn, wmn[...],
                      preferred_element_type=jnp.float32) + bmn[...]


def _fin_body(h, parts, degp, wut, wub, bub, whyw, whyb, yf, loss_o):
  h3 = _update(h, parts, degp, wut, wub, bub)
  g = lax.broadcasted_iota(jnp.int32, (100, N), 0)
  i = lax.broadcasted_iota(jnp.int32, (100, N), 1)
  sel = (i // (N // 10) == g // 10) & (i % 10 == g % 10)
  m = jnp.where(sel, jnp.float32(0.01), jnp.float32(0.0))
  hr = jnp.dot(m, h3, preferred_element_type=jnp.float32)
  yh = jnp.dot(hr, whyw[...], preferred_element_type=jnp.float32) + whyb[...]
  loss_o[...] = jnp.mean(jnp.abs(yh - yf[...]), keepdims=True)


_f32 = jnp.float32

_k0 = pl.pallas_call(
    _k0_body,
    out_shape=(jax.ShapeDtypeStruct((N, DIM), _f32),
               jax.ShapeDtypeStruct((N, DIM), _f32)))

_upd = pl.pallas_call(
    _upd_body,
    out_shape=(jax.ShapeDtypeStruct((N, DIM), _f32),
               jax.ShapeDtypeStruct((N, DIM), _f32)))

_fin = pl.pallas_call(
    _fin_body,
    out_shape=jax.ShapeDtypeStruct((1, 1), _f32))

_spmm = _make_spmm()
_deg = _make_deg()


def kernel(coord, edge_index, y, Wxz_w, Wxz_b, Wm, bm, Wu, bu, Why_w, Why_b):
  src = edge_index[0]
  dst = edge_index[1]
  coordp = jnp.pad(coord, ((0, 0), (0, 6)))
  wxzp = jnp.pad(Wxz_w, ((0, 6), (0, 0)))
  z128 = jnp.zeros((AGGR, DIM), _f32)
  ones128 = jnp.ones((CHUNK, DIM), _f32)

  degp = _deg(dst, ones128, z128)
  h, t = _k0(coordp, wxzp, Wxz_b.reshape(1, DIM),
             Wm[0], bm[0].reshape(1, DIM))

  for l in range(3):
    parts = _spmm(t, src, dst, z128)
    wut = Wu[l][:DIM]
    wub = Wu[l][DIM:]
    bub = bu[l].reshape(1, DIM)
    if l < 2:
      h, t = _upd(h, parts, degp, wut, wub, bub,
                  Wm[l + 1], bm[l + 1].reshape(1, DIM))
    else:
      loss = _fin(h, parts, degp, wut, wub, bub,
                  Why_w, Why_b.reshape(1, 1), y.reshape(100, 1))
  return loss.reshape(())


# speedup vs baseline: 5.0052x; 1.3018x over previous
import functools

import jax
import jax.numpy as jnp
from jax import lax
from jax.experimental import pallas as pl
from jax.experimental.pallas import tpu as pltpu
from jax.experimental.pallas import tpu_sc as plsc

N = 10000
E = 320000
DIM = 128
NC = 2
NS = 16
SPLIT = 5120
SZ1 = N - SPLIT
AGGR = SPLIT + 8
CHUNK = 80
EPT = E // NS
NCHUNK = EPT // CHUNK
RING = 5


def _clamp_dst(dst_t, i, dstc_r, b, half_lo, half_sz):
  for k in range(CHUNK // 16):
    dv = dst_t[pl.ds(i * CHUNK + k * 16, 16)]
    local = dv - half_lo
    ok = (local >= 0) & (local < half_sz)
    dstc_r[b, pl.ds(k * 16, 16)] = jnp.where(ok, local, SPLIT)


def _spmm_body(t_hbm, src_hbm, dst_hbm, z128, out_hbm,
               src_r, dst_t, dstc_r, rows_r, agg_sh, *sems):
  gsem = sems[0:RING]
  ssem = sems[RING:2 * RING]
  isem = sems[2 * RING:3 * RING]

  c = lax.axis_index("c")
  s = lax.axis_index("s")
  ebase = s * EPT
  half_sz = jnp.where(c == 0, SPLIT, SZ1)
  half_lo = c * SPLIT

  @pl.when(s == 0)
  def _zero():
    pltpu.sync_copy(z128, agg_sh)

  pltpu.sync_copy(dst_hbm.at[pl.ds(ebase, EPT)], dst_t)
  plsc.subcore_barrier()

  def _load_src(i, b, p):
    pltpu.async_copy(src_hbm.at[pl.ds(ebase + i * CHUNK, CHUNK)],
                     src_r.at[b, p], isem[b])

  def _gather(b, p):
    pltpu.async_copy(t_hbm.at[src_r.at[b, p]], rows_r.at[b], gsem[b])

  for b in range(RING):
    _load_src(b, b, 0)
  for b in range(RING):
    pltpu.make_async_copy(src_hbm.at[pl.ds(0, CHUNK)], src_r.at[b, 0],
                          isem[b]).wait()
    _gather(b, 0)
    _load_src(RING + b, b, 1)

  def _outer(io, carry):
    for p in range(2):
      for b in range(RING):
        i = io * 2 * RING + p * RING + b
        pltpu.make_async_copy(t_hbm.at[src_r.at[b, p]], rows_r.at[b],
                              gsem[b]).wait()
        _clamp_dst(dst_t, i, dstc_r, b, half_lo, half_sz)
        pltpu.async_copy(rows_r.at[b], agg_sh.at[dstc_r.at[b]], ssem[b],
                         add=True)
        pltpu.make_async_copy(rows_r.at[b], agg_sh.at[dstc_r.at[b]],
                              ssem[b]).wait()

        j = i + RING
        @pl.when(j < NCHUNK)
        def _next_gather():
          pltpu.make_async_copy(src_hbm.at[pl.ds(0, CHUNK)],
                                src_r.at[b, 1 - p], isem[b]).wait()
          _gather(b, 1 - p)

        j2 = i + 2 * RING
        @pl.when(j2 < NCHUNK)
        def _next_load():
          _load_src(j2, b, p)
    return carry

  lax.fori_loop(0, NCHUNK // (2 * RING), _outer, 0)
  plsc.subcore_barrier()
  _writeout_halves(agg_sh, out_hbm, c, s)


def _writeout_halves(sh, hbm, c, s):
  @pl.when((s == 0) & (c == 0))
  def _wo0():
    pltpu.sync_copy(sh.at[pl.ds(0, SPLIT)], hbm.at[pl.ds(0, SPLIT)])

  @pl.when((s == 0) & (c == 1))
  def _wo1():
    pltpu.sync_copy(sh.at[pl.ds(0, SZ1)], hbm.at[pl.ds(SPLIT, SZ1)])


def _deg_body(dst_hbm, ones_hbm, z128, deg_hbm, dst_t, dstc_r, ones_v,
              deg_sh, *ssem):
  c = lax.axis_index("c")
  s = lax.axis_index("s")
  ebase = s * EPT
  half_sz = jnp.where(c == 0, SPLIT, SZ1)
  half_lo = c * SPLIT

  @pl.when(s == 0)
  def _zero():
    pltpu.sync_copy(z128, deg_sh)

  pltpu.sync_copy(ones_hbm, ones_v)
  pltpu.sync_copy(dst_hbm.at[pl.ds(ebase, EPT)], dst_t)
  plsc.subcore_barrier()

  def _one_chunk(i, b):
    _clamp_dst(dst_t, i, dstc_r, b, half_lo, half_sz)
    pltpu.async_copy(ones_v, deg_sh.at[dstc_r.at[b]], ssem[b], add=True)

  for b in range(RING):
    _one_chunk(b, b)

  def _outer(io, carry):
    for b in range(RING):
      i = io * RING + b
      pltpu.make_async_copy(ones_v, deg_sh.at[dstc_r.at[b]], ssem[b]).wait()
      _one_chunk(i, b)
    return carry

  lax.fori_loop(1, NCHUNK // RING, _outer, 0)

  for b in range(RING):
    pltpu.make_async_copy(ones_v, deg_sh.at[dstc_r.at[b]], ssem[b]).wait()

  plsc.subcore_barrier()
  _writeout_halves(deg_sh, deg_hbm, c, s)


def _sc_mesh():
  return plsc.VectorSubcoreMesh(core_axis_name="c", subcore_axis_name="s")


def _make_spmm():
  full = jax.ShapeDtypeStruct((N, DIM), jnp.float32)
  scratch = [
      pltpu.VMEM((RING, 2, CHUNK), jnp.int32),
      pltpu.VMEM((EPT,), jnp.int32),
      pltpu.VMEM((RING, CHUNK), jnp.int32),
      pltpu.VMEM((RING, CHUNK, DIM), jnp.float32),
      pltpu.VMEM_SHARED((AGGR, DIM), jnp.float32),
  ] + [pltpu.SemaphoreType.DMA] * (3 * RING)
  return pl.kernel(_spmm_body, out_type=full, mesh=_sc_mesh(),
                   scratch_types=scratch)


def _make_deg():
  full = jax.ShapeDtypeStruct((N, DIM), jnp.float32)
  scratch = [
      pltpu.VMEM((EPT,), jnp.int32),
      pltpu.VMEM((RING, CHUNK), jnp.int32),
      pltpu.VMEM((CHUNK, DIM), jnp.float32),
      pltpu.VMEM_SHARED((AGGR, DIM), jnp.float32),
  ] + [pltpu.SemaphoreType.DMA] * RING
  return pl.kernel(_deg_body, out_type=full, mesh=_sc_mesh(),
                   scratch_types=scratch)


def _k0_body(coordp, wxzp, bxz, wm0, bm0, h0_o, t0_o):
  h0 = jnp.dot(coordp[...], wxzp[...],
               preferred_element_type=jnp.float32) + bxz[...]
  h0_o[...] = h0
  t0_o[...] = jnp.dot(h0, wm0[...],
                      preferred_element_type=jnp.float32) + bm0[...]


def _update(h, parts, degp, wut, wub, bub):
  inv = 1.0 / jnp.maximum(degp[:, 0:1], 1.0)
  agg = parts[...] * inv
  u = jnp.dot(h[...], wut[...], preferred_element_type=jnp.float32)
  u += jnp.dot(agg, wub[...], preferred_element_type=jnp.float32)
  u = jnp.maximum(u + bub[...], 0.0)
  return h[...] + u


def _upd_body(h, parts, degp, wut, wub, bub, wmn, bmn, hn_o, tn_o):
  hn = _update(h, parts, degp, wut, wub, bub)
  hn_o[...] = hn
  tn_o[...] = jnp.dot(hn, wmn[...],
                      preferred_element_type=jnp.float32) + bmn[...]


def _fin_body(h, parts, degp, wut, wub, bub, whyw, whyb, yf, loss_o):
  h3 = _update(h, parts, degp, wut, wub, bub)
  g = lax.broadcasted_iota(jnp.int32, (100, N), 0)
  i = lax.broadcasted_iota(jnp.int32, (100, N), 1)
  sel = (i // (N // 10) == g // 10) & (i % 10 == g % 10)
  m = jnp.where(sel, jnp.float32(0.01), jnp.float32(0.0))
  hr = jnp.dot(m, h3, preferred_element_type=jnp.float32)
  yh = jnp.dot(hr, whyw[...], preferred_element_type=jnp.float32) + whyb[...]
  loss_o[...] = jnp.mean(jnp.abs(yh - yf[...]), keepdims=True)


_f32 = jnp.float32

_k0 = pl.pallas_call(
    _k0_body,
    out_shape=(jax.ShapeDtypeStruct((N, DIM), _f32),
               jax.ShapeDtypeStruct((N, DIM), _f32)))

_upd = pl.pallas_call(
    _upd_body,
    out_shape=(jax.ShapeDtypeStruct((N, DIM), _f32),
               jax.ShapeDtypeStruct((N, DIM), _f32)))

_fin = pl.pallas_call(
    _fin_body,
    out_shape=jax.ShapeDtypeStruct((1, 1), _f32))

_spmm = _make_spmm()
_deg = _make_deg()


def kernel(coord, edge_index, y, Wxz_w, Wxz_b, Wm, bm, Wu, bu, Why_w, Why_b):
  src = edge_index[0]
  dst = edge_index[1]
  coordp = jnp.pad(coord, ((0, 0), (0, 6)))
  wxzp = jnp.pad(Wxz_w, ((0, 6), (0, 0)))
  z128 = jnp.zeros((AGGR, DIM), _f32)
  ones128 = jnp.ones((CHUNK, DIM), _f32)

  degp = _deg(dst, ones128, z128)
  h, t = _k0(coordp, wxzp, Wxz_b.reshape(1, DIM),
             Wm[0], bm[0].reshape(1, DIM))

  for l in range(3):
    parts = _spmm(t, src, dst, z128)
    wut = Wu[l][:DIM]
    wub = Wu[l][DIM:]
    bub = bu[l].reshape(1, DIM)
    if l < 2:
      h, t = _upd(h, parts, degp, wut, wub, bub,
                  Wm[l + 1], bm[l + 1].reshape(1, DIM))
    else:
      loss = _fin(h, parts, degp, wut, wub, bub,
                  Why_w, Why_b.reshape(1, 1), y.reshape(100, 1))
  return loss.reshape(())
